# Initial kernel scaffold; baseline (speedup 1.0000x reference)
import jax
import jax.numpy as jnp
from jax.experimental import pallas as pl


def kernel(normal_weights, areas, mean_curvatures, positions, attention_mask, batch_sizes, target_normals, target_curvatures):
    raise NotImplementedError("write your pallas kernel here")


# speedup vs baseline: 10.9997x; 10.9997x over previous
import functools

import jax
import jax.numpy as jnp
from jax import lax
from jax.experimental import pallas as pl
from jax.experimental.pallas import tpu as pltpu
from jax.experimental.pallas import tpu_sc as plsc

EPS = 1e-08
NC = 2
NS = 16
NW = NC * NS
LANES = 16


@functools.lru_cache(maxsize=None)
def _build(B, K, TOTAL):
    RPW = B // NW
    C = 128
    NCH = RPW // C
    GRP = C // LANES
    SLAB = C * (K - 1) * 3 + 64
    LIMIT = SLAB - 3

    mesh = plsc.VectorSubcoreMesh(core_axis_name="c", subcore_axis_name="s",
                                  num_cores=NC, num_subcores=NS)

    @functools.partial(
        pl.kernel,
        out_type=jax.ShapeDtypeStruct((B * 3,), jnp.float32),
        mesh=mesh,
        compiler_params=pltpu.CompilerParams(needs_layout_passes=False),
        scratch_types=[
            pltpu.VMEM((RPW,), jnp.int32),
            pltpu.VMEM((RPW,), jnp.int32),
            pltpu.VMEM((RPW,), jnp.float32),
            pltpu.VMEM((C * K,), jnp.float32),
            pltpu.VMEM((SLAB,), jnp.float32),
            pltpu.VMEM((C * 3,), jnp.float32),
        ],
    )
    def ragged(w_hbm, pos_hbm, starts_hbm, sizes_hbm, areas_hbm, out_hbm,
               starts_v, sizes_v, areas_v, wbuf, pbuf, obuf):
        wid = lax.axis_index("s") * NC + lax.axis_index("c")
        r0 = wid * RPW
        pltpu.sync_copy(starts_hbm.at[pl.ds(r0, RPW)], starts_v)
        pltpu.sync_copy(sizes_hbm.at[pl.ds(r0, RPW)], sizes_v)
        pltpu.sync_copy(areas_hbm.at[pl.ds(r0, RPW)], areas_v)
        lane = lax.iota(jnp.int32, LANES)

        for ch in range(NCH):
            pltpu.sync_copy(w_hbm.at[pl.ds((r0 + ch * C) * K, C * K)], wbuf)
            sstart = starts_v[pl.ds(ch * C, LANES)][0]
            base3 = sstart * 3
            base8 = pl.multiple_of(lax.bitwise_and(base3, -8), 8)
            shift = base3 - base8
            pltpu.sync_copy(pos_hbm.at[pl.ds(base8, SLAB)], pbuf)

            def group(g, _, sstart=sstart, shift=shift, ch=ch):
                row = ch * C + g * LANES
                st16 = starts_v[pl.ds(row, LANES)]
                k16 = sizes_v[pl.ds(row, LANES)]
                ar16 = areas_v[pl.ds(row, LANES)]
                pbase = (st16 - sstart) * 3 + shift
                widx0 = (g * LANES + lane) * K

                def jb(j, acc):
                    a0, a1, a2 = acc
                    wv = plsc.load_gather(wbuf, [widx0 + j])
                    wm = jnp.where(j < k16, wv, 0.0)
                    pi = jnp.minimum(pbase + j * 3, LIMIT)
                    p0 = plsc.load_gather(pbuf, [pi])
                    p1 = plsc.load_gather(pbuf, [pi + 1])
                    p2 = plsc.load_gather(pbuf, [pi + 2])
                    return (a0 + wm * p0, a1 + wm * p1, a2 + wm * p2)

                z = jnp.zeros((LANES,), jnp.float32)
                a0, a1, a2 = lax.fori_loop(0, K - 1, jb, (z, z, z))
                inv = 1.0 / (ar16 + EPS)
                oi = (g * LANES + lane) * 3
                plsc.store_scatter(obuf, [oi], a0 * inv)
                plsc.store_scatter(obuf, [oi + 1], a1 * inv)
                plsc.store_scatter(obuf, [oi + 2], a2 * inv)
                return 0

            lax.fori_loop(0, GRP, group, 0)
            pltpu.sync_copy(obuf, out_hbm.at[pl.ds((r0 + ch * C) * 3, C * 3)])

    return ragged


def kernel(normal_weights, areas, mean_curvatures, positions, attention_mask,
           batch_sizes, target_normals, target_curvatures):
    B, K = normal_weights.shape
    TOTAL = positions.shape[0]
    cs = jnp.cumsum(batch_sizes, dtype=jnp.int32)
    starts = jnp.concatenate([jnp.zeros((1,), jnp.int32), cs[:-1]])
    fn = _build(B, K, TOTAL)
    out = fn(normal_weights.reshape(-1), positions.reshape(-1), starts,
             batch_sizes, areas)
    return out.reshape(B, 3)
